# Initial kernel scaffold; baseline (speedup 1.0000x reference)
import jax
import jax.numpy as jnp
from jax.experimental import pallas as pl


def kernel(hidden_states, X, W_mlp, W_struct, Wg, W_proj, W1, W2, W3, lamb, theta, edge_block):
    raise NotImplementedError("write your pallas kernel here")


# speedup vs baseline: 8.6641x; 8.6641x over previous
import functools

import jax
import jax.numpy as jnp
from jax import lax
from jax.experimental import pallas as pl
from jax.experimental.pallas import tpu as pltpu
from jax.experimental.pallas import tpu_sc as plsc


def _gate_body(a8_ref, crow_ref, base1_ref, wmlp_ref, wg0_ref, wg1_ref,
               wproj_ref, h_ref, probs_ref, sel_ref, w_ref, cnt_ref):
    i = pl.program_id(0)
    E = 8
    NN = 9

    h = h_ref[...]
    x = jnp.dot(h, wmlp_ref[...], preferred_element_type=jnp.float32)
    x = x * jax.nn.sigmoid(x)
    xg = jnp.dot(x, wg0_ref[...], preferred_element_type=jnp.float32)

    u = []
    for s in range(NN):
        v = base1_ref[s:s + 1, :] + crow_ref[s] * xg
        u.append(v * jax.nn.sigmoid(v))

    logits_cols = []
    for d in range(E):
        agg = a8_ref[d, 0] * u[0]
        for s in range(1, NN):
            agg = agg + a8_ref[d, s] * u[s]
        z = jnp.dot(agg, wg1_ref[...], preferred_element_type=jnp.float32)
        z = z * jax.nn.sigmoid(z)
        logits_cols.append(
            jnp.dot(z, wproj_ref[...], preferred_element_type=jnp.float32))
    logits = jnp.concatenate(logits_cols, axis=1)

    p = jax.nn.softmax(logits, axis=-1)
    probs_ref[...] = p

    lane = lax.broadcasted_iota(jnp.int32, p.shape, 1)
    m1 = jnp.max(p, axis=1, keepdims=True)
    i1 = jnp.min(jnp.where(p == m1, lane, E), axis=1, keepdims=True)
    p2 = jnp.where(lane == i1, -1.0, p)
    m2 = jnp.max(p2, axis=1, keepdims=True)
    i2 = jnp.min(jnp.where(p2 == m2, lane, E), axis=1, keepdims=True)

    sel_ref[:, 0:1] = i1
    sel_ref[:, 1:2] = i2
    tot = m1 + m2
    w_ref[:, 0:1] = m1 / tot
    w_ref[:, 1:2] = m2 / tot

    oh1 = jnp.where(lane == i1, m1, 0.0)
    oh2 = jnp.where(lane == i2, m2, 0.0)
    contrib = jnp.sum(oh1 + oh2, axis=0, keepdims=True)

    @pl.when(i == 0)
    def _():
        cnt_ref[...] = jnp.zeros_like(cnt_ref)

    cnt_ref[...] += contrib


def _run_gate(h2, a8, crow, base1, w_mlp, wg0, wg1, wproj, T, D, E, BTG):
    grid = (T // BTG,)
    smem = functools.partial(pl.BlockSpec, memory_space=pltpu.SMEM)
    return pl.pallas_call(
        _gate_body,
        grid=grid,
        in_specs=[
            smem(),
            smem(),
            pl.BlockSpec((9, 64), lambda i: (0, 0)),
            pl.BlockSpec((D, 64), lambda i: (0, 0)),
            pl.BlockSpec((64, 64), lambda i: (0, 0)),
            pl.BlockSpec((64, 64), lambda i: (0, 0)),
            pl.BlockSpec((64, 1), lambda i: (0, 0)),
            pl.BlockSpec((BTG, D), lambda i: (i, 0)),
        ],
        out_specs=[
            pl.BlockSpec((BTG, E), lambda i: (i, 0)),
            pl.BlockSpec((BTG, 2), lambda i: (i, 0)),
            pl.BlockSpec((BTG, 2), lambda i: (i, 0)),
            pl.BlockSpec((1, E), lambda i: (0, 0)),
        ],
        out_shape=[
            jax.ShapeDtypeStruct((T, E), jnp.float32),
            jax.ShapeDtypeStruct((T, 2), jnp.int32),
            jax.ShapeDtypeStruct((T, 2), jnp.float32),
            jax.ShapeDtypeStruct((1, E), jnp.float32),
        ],
    )(a8, crow, base1, w_mlp, wg0, wg1, wproj, h2)


def _ffn_body(be_ref, nv_ref, h_ref, wgt_ref, w1_ref, w3_ref, w2_ref, out_ref):
    i = pl.program_id(0)

    @pl.when(i < nv_ref[0])
    def _():
        h = h_ref[...]
        a1 = jnp.dot(h, w1_ref[0], preferred_element_type=jnp.float32)
        a3 = jnp.dot(h, w3_ref[0], preferred_element_type=jnp.float32)
        g = a1 * jax.nn.sigmoid(a1) * a3
        eo = jnp.dot(g, w2_ref[0], preferred_element_type=jnp.float32)
        out_ref[...] = eo * wgt_ref[...]

    @pl.when(i >= nv_ref[0])
    def _():
        out_ref[...] = jnp.zeros_like(out_ref)


def _run_ffn(h_sorted, wgt_sorted, W1, W3, W2, be, nvalid, P, NB, BT, D, DFF, E):
    grid_spec = pltpu.PrefetchScalarGridSpec(
        num_scalar_prefetch=2,
        grid=(NB,),
        in_specs=[
            pl.BlockSpec((BT, D), lambda i, be, nv: (i, 0)),
            pl.BlockSpec((BT, 1), lambda i, be, nv: (i, 0)),
            pl.BlockSpec((1, D, DFF), lambda i, be, nv: (be[i], 0, 0)),
            pl.BlockSpec((1, D, DFF), lambda i, be, nv: (be[i], 0, 0)),
            pl.BlockSpec((1, DFF, D), lambda i, be, nv: (be[i], 0, 0)),
        ],
        out_specs=pl.BlockSpec((BT, D), lambda i, be, nv: (i, 0)),
    )
    return pl.pallas_call(
        _ffn_body,
        grid_spec=grid_spec,
        out_shape=jax.ShapeDtypeStruct((P, D), jnp.float32),
    )(be, nvalid, h_sorted, wgt_sorted, W1, W3, W2)


def _make_sc_gather(T, D, P, NW, NC, CH):
    rows_per_w = P // NW
    mesh = plsc.VectorSubcoreMesh(core_axis_name="c", subcore_axis_name="s")

    @functools.partial(
        pl.kernel,
        mesh=mesh,
        out_type=jax.ShapeDtypeStruct((P, D), jnp.float32),
        scratch_types=[
            pltpu.VMEM((CH,), jnp.int32),
            pltpu.VMEM((CH, D), jnp.float32),
            pltpu.SemaphoreType.DMA,
        ],
    )
    def k(h_hbm, tok_hbm, out_hbm, idx_v, rows_v, sem):
        wid = lax.axis_index("s") * NC + lax.axis_index("c")
        base = wid * rows_per_w

        def chunk(j, carry):
            off = base + j * CH
            pltpu.sync_copy(tok_hbm.at[pl.ds(off, CH)], idx_v)
            pltpu.async_copy(h_hbm.at[idx_v], rows_v, sem).wait()
            pltpu.sync_copy(rows_v, out_hbm.at[pl.ds(off, CH)])
            return carry

        lax.fori_loop(0, rows_per_w // CH, chunk, 0)

    return k


def _make_sc_combine(T, D, P, NW, NC, CH):
    rows_per_w = T // NW
    mesh = plsc.VectorSubcoreMesh(core_axis_name="c", subcore_axis_name="s")

    @functools.partial(
        pl.kernel,
        mesh=mesh,
        out_type=jax.ShapeDtypeStruct((T, D), jnp.float32),
        scratch_types=[
            pltpu.VMEM((CH,), jnp.int32),
            pltpu.VMEM((CH, D), jnp.float32),
            pltpu.VMEM((CH, D), jnp.float32),
            pltpu.SemaphoreType.DMA,
        ],
    )
    def k(outs_hbm, inv0_hbm, inv1_hbm, final_hbm, idx_v, rows_a, rows_b, sem):
        wid = lax.axis_index("s") * NC + lax.axis_index("c")
        base = wid * rows_per_w

        def chunk(j, carry):
            off = base + j * CH
            pltpu.sync_copy(inv0_hbm.at[pl.ds(off, CH)], idx_v)
            pltpu.async_copy(outs_hbm.at[idx_v], rows_a, sem).wait()
            pltpu.sync_copy(inv1_hbm.at[pl.ds(off, CH)], idx_v)
            pltpu.async_copy(outs_hbm.at[idx_v], rows_b, sem).wait()

            def row_add(r, c2):
                for cc in range(D // 16):
                    sl = pl.ds(cc * 16, 16)
                    rows_a[r, sl] = rows_a[r, sl] + rows_b[r, sl]
                return c2

            lax.fori_loop(0, CH, row_add, 0)
            pltpu.sync_copy(rows_a, final_hbm.at[pl.ds(off, CH)])
            return carry

        lax.fori_loop(0, rows_per_w // CH, chunk, 0)

    return k


def kernel(hidden_states, X, W_mlp, W_struct, Wg, W_proj, W1, W2, W3,
           lamb, theta, edge_block):
    bs, sl, D = hidden_states.shape
    T = bs * sl
    E, _, DFF = W1.shape
    K = 2
    NN = E + 1
    BT = 256
    BTG = 512
    TK = T * K
    P = TK + E * BT
    NB = P // BT
    NW, NC, CH = 32, 2, 64

    h2 = hidden_states.reshape(T, D)

    src = edge_block[0].astype(jnp.int32)
    dst = edge_block[1].astype(jnp.int32)
    eye = jnp.eye(NN, dtype=jnp.float32)
    C = eye + jnp.zeros((NN, NN), jnp.float32).at[dst, src].add(1.0)
    deg = jnp.sum(C, axis=1)
    dinv = deg ** -0.5
    A = C * dinv[:, None] * dinv[None, :]

    expv = X @ W_struct
    expv = expv * jax.nn.sigmoid(expv)
    base1 = (A[:, :E] @ expv) @ Wg[0]
    crow = A[:, E]
    a8 = A[:E, :]

    probs, sel, wn, cnt = _run_gate(
        h2, a8, crow, base1, W_mlp, Wg[0], Wg[1], W_proj, T, D, E, BTG)

    e_flat = sel.reshape(TK)
    w_flat = wn.reshape(TK)
    oh = (e_flat[:, None] == jnp.arange(E, dtype=jnp.int32)[None, :])
    ranks = jnp.cumsum(oh.astype(jnp.int32), axis=0)
    counts = ranks[-1]
    nb_e = (counts + BT - 1) // BT
    cum_nb = jnp.cumsum(nb_e)
    poff = (jnp.concatenate([jnp.zeros((1,), cum_nb.dtype), cum_nb[:-1]])
            * BT)
    rank_p = jnp.take_along_axis(ranks, e_flat[:, None], axis=1)[:, 0]
    pos = (poff[e_flat] + rank_p - 1).astype(jnp.int32)

    order = jnp.argsort(e_flat, stable=True)
    off_e = jnp.concatenate([jnp.zeros((1,), counts.dtype),
                             jnp.cumsum(counts)[:-1]])
    slot = jnp.arange(P, dtype=jnp.int32)
    blk = slot // BT
    be_full = jnp.searchsorted(cum_nb, jnp.arange(NB), side='right')
    be = jnp.minimum(be_full, E - 1).astype(jnp.int32)
    e_of_slot = be[blk]
    j_in_e = slot - poff[e_of_slot].astype(jnp.int32)
    valid = j_in_e < counts[e_of_slot]
    q = jnp.clip(off_e[e_of_slot].astype(jnp.int32) + j_in_e, 0, TK - 1)
    pair_at_slot = order[q].astype(jnp.int32)
    tok_sorted = jnp.where(valid, pair_at_slot // K, 0).astype(jnp.int32)
    wgt_sorted = jnp.where(valid, w_flat[pair_at_slot], 0.0)[:, None]
    nvalid = (cum_nb[-1]).astype(jnp.int32).reshape(1)
    inv = pos.reshape(T, K)

    h_sorted = _make_sc_gather(T, D, P, NW, NC, CH)(h2, tok_sorted)

    outs = _run_ffn(h_sorted, wgt_sorted, W1, W3, W2, be, nvalid,
                    P, NB, BT, D, DFF, E)

    final2 = _make_sc_combine(T, D, P, NW, NC, CH)(
        outs, inv[:, 0].astype(jnp.int32), inv[:, 1].astype(jnp.int32))

    final = final2.reshape(bs, sl, D)
    loss_component = jnp.concatenate([
        probs,
        (cnt * float(K)),
        jnp.broadcast_to(lamb.reshape(1, 1), (1, E)),
        jnp.broadcast_to(theta.reshape(1, 1), (1, E)),
    ], axis=0)
    return final, loss_component
